# Initial kernel scaffold; baseline (speedup 1.0000x reference)
import jax
import jax.numpy as jnp
from jax.experimental import pallas as pl


def kernel(input, w1, b1, a1, w2, b2, a2, w3, b3, a3):
    raise NotImplementedError("write your pallas kernel here")


# speedup vs baseline: 3.1755x; 3.1755x over previous
import functools

import jax
import jax.numpy as jnp
from jax import lax
from jax.experimental import pallas as pl
from jax.experimental.pallas import tpu as pltpu
from jax.experimental.pallas import tpu_sc as plsc

N = 2
C = 64
HW = 512 * 512
NSEL = 900
NPAD = 1024
NTILES = 16
CHUNK = HW // NTILES
NBINS = 65
HROW = 80
CPC = 4

def _mesh():
    return plsc.VectorSubcoreMesh(core_axis_name="c", subcore_axis_name="s")


def _z_body(x_ref, z_ref):
    x = x_ref[...]
    z_ref[...] = jnp.sum((x >= 0).astype(jnp.int32), axis=1)


_ZBLK = 4096


def _compute_z(x):
    return pl.pallas_call(
        _z_body,
        grid=(HW // _ZBLK,),
        in_specs=[pl.BlockSpec((N, C, _ZBLK), lambda i: (0, 0, i))],
        out_specs=pl.BlockSpec((N, _ZBLK), lambda i: (0, i)),
        out_shape=jax.ShapeDtypeStruct((N, HW), jnp.int32),
    )(x)


def _select_gather_body(z_hbm, xflat_hbm, cp_hbm, q_hbm,
                        zbuf, histbuf, histpub, hist_all, gbuf, cbuf,
                        cpbuf, slotbuf, cploc, idxbuf, valbuf,
                        sh_hist, sh_cp, gsem):
    b = lax.axis_index("c")
    s = lax.axis_index("s")
    iota = lax.iota(jnp.int32, 16)
    zero16 = jnp.zeros((16,), jnp.int32)
    ones16 = jnp.ones((16,), jnp.int32)

    pltpu.sync_copy(z_hbm.at[b, pl.ds(s * CHUNK, CHUNK)], zbuf)

    for k in range(16 * HROW // 16):
        histbuf[pl.ds(k * 16, 16)] = zero16
    lane_off = iota * HROW

    def _hist_step(i, _):
        v = zbuf[pl.ds(i * 16, 16)]
        plsc.addupdate_scatter(histbuf, [v + lane_off], ones16)
        return 0

    lax.fori_loop(0, CHUNK // 16, _hist_step, 0)

    for k in range(HROW // 16):
        acc = zero16
        for l in range(16):
            acc = acc + histbuf[pl.ds(l * HROW + k * 16, 16)]
        histpub[pl.ds(k * 16, 16)] = acc

    pltpu.sync_copy(histpub, sh_hist.at[s])
    plsc.subcore_barrier()
    pltpu.sync_copy(sh_hist, hist_all)

    total = jnp.int32(0)
    for k in range(HROW // 16):
        g = zero16
        for l in range(16):
            g = g + hist_all[l, pl.ds(k * 16, 16)]
        cums = plsc.cumsum(g) + total
        gbuf[pl.ds(k * 16, 16)] = g
        cbuf[pl.ds(k * 16, 16)] = cums
        total = total + jnp.sum(g, axis=0)

    T = jnp.int32(9999)
    for k in range(HROW // 16):
        cums = cbuf[pl.ds(k * 16, 16)]
        f = plsc.all_reduce_ffs(cums >= NSEL)
        fs = jnp.max(f, axis=0)
        cand = k * 16 + fs
        T = jnp.where((T == 9999) & (fs < 16), cand, T)

    tvec = zero16 + T
    gT = jnp.max(plsc.load_gather(gbuf, [tvec]), axis=0)
    cT = jnp.max(plsc.load_gather(cbuf, [tvec]), axis=0)
    m_quota = NSEL - (cT - gT)

    prefix_lt = jnp.int32(0)
    prefixT = jnp.int32(0)
    for l in range(16):
        cl = jnp.int32(0)
        for k in range(HROW // 16):
            hv = hist_all[l, pl.ds(k * 16, 16)]
            vidx = iota + (k * 16)
            cl = cl + jnp.sum(jnp.where(vidx < T, hv, 0), axis=0)
        lvec = zero16 + l
        ctl = jnp.max(plsc.load_gather(hist_all, [lvec, tvec]), axis=0)
        before = jnp.where(jnp.int32(l) < s, jnp.int32(1), jnp.int32(0))
        prefix_lt = prefix_lt + before * cl
        prefixT = prefixT + before * ctl
    quota = m_quota - prefixT
    base = prefix_lt + jnp.minimum(m_quota, prefixT)

    def _scan_step(i, carry):
        off, eqc = carry
        v = zbuf[pl.ds(i * 16, 16)]
        lt = v < T
        eq = v == T
        eqi = jnp.where(eq, jnp.int32(1), jnp.int32(0))
        eqcs = plsc.cumsum(eqi)
        sel = lt | (eq & ((eqc + eqcs) <= quota))
        gidx = (s * CHUNK + i * 16) + iota
        plsc.store_compressed(cpbuf.at[pl.ds(off, 16)], gidx, mask=sel)
        npop = jnp.max(plsc.all_reduce_population_count(sel), axis=0)
        return off + npop, eqc + jnp.sum(eqi, axis=0)

    mycnt, _ = lax.fori_loop(0, CHUNK // 16, _scan_step,
                             (jnp.int32(0), jnp.int32(0)))

    for j in range(NPAD // 16):
        jv = j * 16 + iota
        slot = jnp.where(jv < mycnt, base + jv, NPAD + iota)
        slotbuf[j // 8, pl.ds((j % 8) * 16, 16)] = slot
    for j in range(8):
        pltpu.sync_copy(cpbuf.at[pl.ds(j * 128, 128)],
                        sh_cp.at[slotbuf.at[j]])
    plsc.subcore_barrier()

    @pl.when(s == 0)
    def _():
        pltpu.sync_copy(sh_cp.at[pl.ds(0, NPAD)], cploc)
        cp0 = plsc.load_gather(cploc, [zero16])
        v896 = cploc[pl.ds(896, 16)]
        cploc[pl.ds(896, 16)] = jnp.where(iota < 4, v896, cp0)
        for k in range(57, NPAD // 16):
            cploc[pl.ds(k * 16, 16)] = cp0
        pltpu.sync_copy(cploc, sh_cp.at[pl.ds(0, NPAD)])
        pltpu.sync_copy(cploc, cp_hbm.at[b])

    plsc.subcore_barrier()

    pltpu.sync_copy(sh_cp.at[pl.ds(0, NPAD)], cploc)
    for ci in range(CPC):
        ch = s * CPC + ci
        chbase = (b * C + ch) * HW
        for j in range(8):
            for v in range(8):
                idxbuf[ci * 8 + j, pl.ds(v * 16, 16)] = (
                    cploc[pl.ds(j * 128 + v * 16, 16)] + chbase)
    handles = []
    for ci in range(CPC):
        for j in range(8):
            handles.append(pltpu.async_copy(
                xflat_hbm.at[idxbuf.at[ci * 8 + j]],
                valbuf.at[ci, pl.ds(j * 128, 128)], gsem))
    for h in handles:
        h.wait()
    for ci in range(CPC):
        ch = s * CPC + ci
        pltpu.sync_copy(valbuf.at[ci], q_hbm.at[b, ch])


def _select_gather(z, xflat):
    kfn = pl.kernel(
        _select_gather_body,
        out_type=(jax.ShapeDtypeStruct((N, NPAD), jnp.int32),
                  jax.ShapeDtypeStruct((N, C, NPAD), jnp.float32)),
        mesh=_mesh(),
        scratch_types=[
            pltpu.VMEM((CHUNK,), jnp.int32),
            pltpu.VMEM((16 * HROW,), jnp.int32),
            pltpu.VMEM((HROW,), jnp.int32),
            pltpu.VMEM((16, HROW), jnp.int32),
            pltpu.VMEM((HROW,), jnp.int32),
            pltpu.VMEM((HROW,), jnp.int32),
            pltpu.VMEM((NPAD,), jnp.int32),
            pltpu.VMEM((8, 128), jnp.int32),
            pltpu.VMEM((NPAD,), jnp.int32),
            pltpu.VMEM((CPC * 8, 128), jnp.int32),
            pltpu.VMEM((CPC, NPAD), jnp.float32),
            pltpu.VMEM_SHARED((16, HROW), jnp.int32),
            pltpu.VMEM_SHARED((NPAD + 16,), jnp.int32),
            pltpu.SemaphoreType.DMA,
        ],
        compiler_params=pltpu.CompilerParams(needs_layout_passes=False),
    )
    return kfn(z, xflat)


def _attn_body(q_ref, w1_ref, b1_ref, a1_ref, w2_ref, b2_ref, a2_ref,
               w3_ref, b3_ref, a3_ref, o_ref):
    q = q_ref[0]

    def conv_prelu(w_ref, b_ref, a_ref):
        y = lax.dot_general(w_ref[...], q, (((1,), (0,)), ((), ())),
                            preferred_element_type=jnp.float32)
        y = y + b_ref[...]
        a = a_ref[0, 0]
        return jnp.where(y >= 0, y, a * y)

    x1 = conv_prelu(w1_ref, b1_ref, a1_ref)
    x2 = conv_prelu(w2_ref, b2_ref, a2_ref)
    xa = conv_prelu(w3_ref, b3_ref, a3_ref)

    score = lax.dot_general(x1, x2, (((0,), (0,)), ((), ())),
                            preferred_element_type=jnp.float32)
    colmask = lax.broadcasted_iota(jnp.int32, (1, NPAD), 1) < NSEL
    score = jnp.where(colmask, score, -1e30)
    mx = jnp.max(score, axis=1, keepdims=True)
    p = jnp.exp(score - mx)
    probs = p / jnp.sum(p, axis=1, keepdims=True)
    attn = lax.dot_general(xa, probs, (((1,), (1,)), ((), ())),
                           preferred_element_type=jnp.float32)
    o_ref[0] = attn + q


def _attention(q, w1, b1, a1, w2, b2, a2, w3, b3, a3):
    full = lambda shape: pl.BlockSpec(shape, lambda i: tuple(0 for _ in shape))
    return pl.pallas_call(
        _attn_body,
        grid=(N,),
        in_specs=[
            pl.BlockSpec((1, C, NPAD), lambda i: (i, 0, 0)),
            full((C // 2, C)), full((C // 2, 1)),
            pl.BlockSpec(memory_space=pltpu.SMEM),
            full((C // 2, C)), full((C // 2, 1)),
            pl.BlockSpec(memory_space=pltpu.SMEM),
            full((C, C)), full((C, 1)),
            pl.BlockSpec(memory_space=pltpu.SMEM),
        ],
        out_specs=pl.BlockSpec((1, C, NPAD), lambda i: (i, 0, 0)),
        out_shape=jax.ShapeDtypeStruct((N, C, NPAD), jnp.float32),
    )(q, w1, b1.reshape(C // 2, 1), a1.reshape(1, 1),
      w2, b2.reshape(C // 2, 1), a2.reshape(1, 1),
      w3, b3.reshape(C, 1), a3.reshape(1, 1))


_CPW = C * HW // NTILES
_CCH = 16384
_NCH = _CPW // _CCH
_NBUF = 4


def _copy_scatter_body(xflat_hbm, ni_hbm, cp_hbm, oflat_hbm,
                       bufs, cploc, idxbuf, valbuf,
                       isem0, isem1, isem2, isem3,
                       osem0, osem1, osem2, osem3, ssem):
    b = lax.axis_index("c")
    s = lax.axis_index("s")
    isems = [isem0, isem1, isem2, isem3]
    osems = [osem0, osem1, osem2, osem3]

    tbase = b * (C * HW) + s * _CPW
    out_handles = [None] * _NBUF
    for k in range(_NCH):
        slot = k % _NBUF
        if out_handles[slot] is not None:
            out_handles[slot].wait()
        off = tbase + k * _CCH
        pltpu.async_copy(xflat_hbm.at[pl.ds(off, _CCH)], bufs.at[slot],
                         isems[slot]).wait()
        out_handles[slot] = pltpu.async_copy(
            bufs.at[slot], oflat_hbm.at[pl.ds(off, _CCH)], osems[slot])
    for h in out_handles:
        if h is not None:
            h.wait()
    plsc.subcore_barrier()

    pltpu.sync_copy(cp_hbm.at[b], cploc)
    for ci in range(CPC):
        ch = s * CPC + ci
        chbase = (b * C + ch) * HW
        pltpu.sync_copy(ni_hbm.at[b, ch], valbuf.at[ci])
        for j in range(8):
            for v in range(8):
                idxbuf[ci * 8 + j, pl.ds(v * 16, 16)] = (
                    cploc[pl.ds(j * 128 + v * 16, 16)] + chbase)
    handles = []
    for ci in range(CPC):
        for j in range(8):
            handles.append(pltpu.async_copy(
                valbuf.at[ci, pl.ds(j * 128, 128)],
                oflat_hbm.at[idxbuf.at[ci * 8 + j]], ssem))
    for h in handles:
        h.wait()


def _copy_scatter(xflat, ni, cp):
    kfn = pl.kernel(
        _copy_scatter_body,
        out_type=jax.ShapeDtypeStruct((N * C * HW,), jnp.float32),
        mesh=_mesh(),
        scratch_types=[
            pltpu.VMEM((_NBUF, _CCH), jnp.float32),
            pltpu.VMEM((NPAD,), jnp.int32),
            pltpu.VMEM((CPC * 8, 128), jnp.int32),
            pltpu.VMEM((CPC, NPAD), jnp.float32),
            pltpu.SemaphoreType.DMA, pltpu.SemaphoreType.DMA,
            pltpu.SemaphoreType.DMA, pltpu.SemaphoreType.DMA,
            pltpu.SemaphoreType.DMA, pltpu.SemaphoreType.DMA,
            pltpu.SemaphoreType.DMA, pltpu.SemaphoreType.DMA,
            pltpu.SemaphoreType.DMA,
        ],
        compiler_params=pltpu.CompilerParams(needs_layout_passes=False),
    )
    return kfn(xflat, ni, cp)


def kernel(input, w1, b1, a1, w2, b2, a2, w3, b3, a3):
    x = input.reshape(N, C, HW)
    xflat = input.reshape(N * C * HW)
    z = _compute_z(x)
    cp, q = _select_gather(z, xflat)
    ni = _attention(q, w1, b1, a1, w2, b2, a2, w3, b3, a3)
    out = _copy_scatter(xflat, ni, cp)
    return out.reshape(N, C, HW)
